# baseline (speedup 1.0000x reference)
import jax
import jax.numpy as jnp
from jax import lax
from jax.experimental import pallas as pl
from jax.experimental.pallas import tpu as pltpu
from jax.experimental.pallas import tpu_sc as plsc

NUM_CORES = 2
NUM_SUBCORES = 16
NUM_WORKERS = NUM_CORES * NUM_SUBCORES
LANES = 16

NUM_ROWS = 1000000
BATCH = 16384
DIM = 64
B_PER_W = BATCH // NUM_WORKERS
CHUNK = 128
N_CHUNKS = B_PER_W // CHUNK
N_SLICES = B_PER_W // LANES
NSLOT = 4


def _body(users_hbm, items_hbm, utab_hbm, itab_hbm, out_hbm,
          idx_u, idx_i, ubuf, vbuf, acc, sems):
    wid = lax.axis_index("s") * NUM_CORES + lax.axis_index("c")
    base = wid * B_PER_W

    for c in range(N_CHUNKS):
        pltpu.sync_copy(users_hbm.at[pl.ds(base + c * CHUNK, CHUNK)], idx_u.at[c])
        pltpu.sync_copy(items_hbm.at[pl.ds(base + c * CHUNK, CHUNK)], idx_i.at[c])

    zeros = jnp.zeros((LANES,), jnp.float32)
    for s in range(N_SLICES):
        acc[pl.ds(s * LANES, LANES)] = zeros

    def fire(d, slot):
        off = pl.multiple_of(d * NUM_ROWS, NUM_ROWS)
        for c in range(N_CHUNKS):
            pltpu.async_copy(
                utab_hbm.at[pl.ds(off, NUM_ROWS)].at[idx_u.at[c]],
                ubuf.at[slot, pl.ds(c * CHUNK, CHUNK)], sems[slot])
            pltpu.async_copy(
                itab_hbm.at[pl.ds(off, NUM_ROWS)].at[idx_i.at[c]],
                vbuf.at[slot, pl.ds(c * CHUNK, CHUNK)], sems[slot])

    def drain(slot):
        for c in range(N_CHUNKS):
            pltpu.make_async_copy(
                utab_hbm.at[pl.ds(0, NUM_ROWS)].at[idx_u.at[c]],
                ubuf.at[slot, pl.ds(c * CHUNK, CHUNK)], sems[slot]).wait()
            pltpu.make_async_copy(
                itab_hbm.at[pl.ds(0, NUM_ROWS)].at[idx_i.at[c]],
                vbuf.at[slot, pl.ds(c * CHUNK, CHUNK)], sems[slot]).wait()

    for d0 in range(NSLOT - 1):
        fire(d0, d0)

    def outer(g, _):
        for j in range(NSLOT):
            d = g * NSLOT + j
            drain(j)
            for s in range(N_SLICES):
                sl = pl.ds(s * LANES, LANES)
                acc[sl] = acc[sl] + ubuf[j, sl] * vbuf[j, sl]

            @pl.when(d + NSLOT - 1 < DIM)
            def _prefetch():
                fire(d + NSLOT - 1, (j + NSLOT - 1) % NSLOT)
        return _

    lax.fori_loop(0, DIM // NSLOT, outer, None)

    for s in range(N_SLICES):
        sl = pl.ds(s * LANES, LANES)
        acc[sl] = 1.0 / (1.0 + jnp.exp(-acc[sl]))

    pltpu.sync_copy(acc, out_hbm.at[pl.ds(base, B_PER_W)])


@jax.jit
def _run(users, items, user_table, item_table):
    uflat = user_table.T.reshape(-1)
    iflat = item_table.T.reshape(-1)
    mesh = plsc.VectorSubcoreMesh(core_axis_name="c", subcore_axis_name="s")
    return pl.kernel(
        _body,
        out_type=jax.ShapeDtypeStruct((BATCH,), jnp.float32),
        mesh=mesh,
        compiler_params=pltpu.CompilerParams(use_tc_tiling_on_sc=False),
        scratch_types=[
            pltpu.VMEM((N_CHUNKS, CHUNK), jnp.int32),
            pltpu.VMEM((N_CHUNKS, CHUNK), jnp.int32),
            pltpu.VMEM((NSLOT, B_PER_W), jnp.float32),
            pltpu.VMEM((NSLOT, B_PER_W), jnp.float32),
            pltpu.VMEM((B_PER_W,), jnp.float32),
            [pltpu.SemaphoreType.DMA] * NSLOT,
        ],
    )(users, items, uflat, iflat)


def kernel(users, items, user_table, item_table):
    return _run(users, items, user_table, item_table)
